# baseline (speedup 1.0000x reference)
import jax
import jax.numpy as jnp
from jax import lax
from jax.experimental import pallas as pl
from jax.experimental.pallas import tpu as pltpu
from jax.experimental.pallas import tpu_sc as plsc

N = 50000
E = 800000
D = 128
H = 64
C = 2
PW = 128

NC = 2
NS = 16
CB = 64
CH = 395
E_PAD = NC * NS * CH * CB
SHARDS = 4
SHARD = 13824
NROWS = SHARDS * SHARD
ACC_ROWS = SHARD + 128
LTRASH = SHARD
ZPT = ACC_ROWS // NS
ZR = 8
WPT = SHARD // NS
TB = 1000
GRID = N // TB

_SC_MESH = plsc.VectorSubcoreMesh(core_axis_name="c", subcore_axis_name="s")


def _segsum_body(hp, srcp, dst4, out, acc, zbuf, src_v, dst_v, rows, sg0, sg1):
    c = lax.axis_index("c")
    s = lax.axis_index("s")
    zv = jnp.zeros((16,), jnp.float32)

    def zrow(i, _):
        for k in range(0, PW, 16):
            zbuf[i, pl.ds(k, 16)] = zv
        return 0

    lax.fori_loop(0, ZR, zrow, 0)

    def shard_pass(p, _):
        for j in range(ZPT // ZR):
            pltpu.sync_copy(zbuf, acc.at[pl.ds(s * ZPT + j * ZR, ZR)])
        plsc.subcore_barrier()

        def fire(chunk, b):
            base = ((c * NS + s) * CH + chunk) * CB
            pltpu.sync_copy(srcp.at[pl.ds(base, CB)], src_v.at[b])
            pltpu.sync_copy(dst4.at[p, pl.ds(base, CB)], dst_v.at[b])
            sem = sg0 if b == 0 else sg1
            pltpu.async_copy(hp.at[src_v.at[b]], rows.at[b], sem)

        def wait(b):
            sem = sg0 if b == 0 else sg1
            pltpu.make_async_copy(hp.at[src_v.at[b]], rows.at[b], sem).wait()

        def scat(b):
            pltpu.sync_copy(rows.at[b], acc.at[dst_v.at[b]], add=True)

        fire(0, 0)

        def body(i, _):
            e = 2 * i
            fire(e + 1, 1)
            wait(0)
            scat(0)
            fire(e + 2, 0)
            wait(1)
            scat(1)
            return 0

        lax.fori_loop(0, (CH - 1) // 2, body, 0)
        wait(0)
        scat(0)

        plsc.subcore_barrier()
        pltpu.sync_copy(acc.at[pl.ds(s * WPT, WPT)],
                        out.at[c, pl.ds(p * SHARD + s * WPT, WPT)])
        plsc.subcore_barrier()
        return 0

    lax.fori_loop(0, SHARDS, shard_pass, 0)


def _segsum(hp, srcp, dst4):
    return pl.kernel(
        _segsum_body,
        out_type=jax.ShapeDtypeStruct((NC, NROWS, PW), jnp.float32),
        mesh=_SC_MESH,
        scratch_types=[
            pltpu.VMEM_SHARED((ACC_ROWS, PW), jnp.float32),
            pltpu.VMEM((ZR, PW), jnp.float32),
            pltpu.VMEM((2, CB), jnp.int32),
            pltpu.VMEM((2, CB), jnp.int32),
            pltpu.VMEM((2, CB, PW), jnp.float32),
            pltpu.SemaphoreType.DMA,
            pltpu.SemaphoreType.DMA,
        ],
    )(hp, srcp, dst4)


def _pack(h):
    col = lax.broadcasted_iota(jnp.int32, (h.shape[0], PW), 1)
    z = jnp.zeros((h.shape[0], PW - H), jnp.float32)
    return jnp.concatenate([h, z], axis=1) + (col == H).astype(jnp.float32)


def _k0_body(x_r, w_r, b_r, hp_r):
    h = jnp.dot(x_r[...], w_r[...], preferred_element_type=jnp.float32)
    hp_r[...] = _pack(jnp.maximum(h + b_r[...], 0.0))


def _k0(x, WinT, b_in2):
    return pl.pallas_call(
        _k0_body,
        grid=(GRID,),
        in_specs=[
            pl.BlockSpec((TB, D), lambda i: (i, 0)),
            pl.BlockSpec((D, H), lambda i: (0, 0)),
            pl.BlockSpec((1, H), lambda i: (0, 0)),
        ],
        out_specs=pl.BlockSpec((TB, PW), lambda i: (i, 0)),
        out_shape=jax.ShapeDtypeStruct((N, PW), jnp.float32),
    )(x, WinT, b_in2)


def _passA_body(acc_r, hp_r, wl_r, bl_r, wr_r, pre_r, st_r):
    i = pl.program_id(0)
    a = acc_r[0] + acc_r[1]
    deg = a[:, H:H + 1]
    agg = a[:, :H] / jnp.maximum(deg, 1.0)
    h = hp_r[:, :H]
    pre = (jnp.dot(agg, wl_r[...], preferred_element_type=jnp.float32)
           + bl_r[...]
           + jnp.dot(h, wr_r[...], preferred_element_type=jnp.float32))
    pre_r[...] = pre

    @pl.when(i == 0)
    def _():
        st_r[...] = jnp.zeros_like(st_r)

    st_r[...] += jnp.stack([jnp.sum(pre, axis=0), jnp.sum(pre * pre, axis=0)])


def _passA(acc2, hp, WlT, bl2, WrT):
    return pl.pallas_call(
        _passA_body,
        grid=(GRID,),
        in_specs=[
            pl.BlockSpec((NC, TB, PW), lambda i: (0, i, 0)),
            pl.BlockSpec((TB, PW), lambda i: (i, 0)),
            pl.BlockSpec((H, H), lambda i: (0, 0)),
            pl.BlockSpec((1, H), lambda i: (0, 0)),
            pl.BlockSpec((H, H), lambda i: (0, 0)),
        ],
        out_specs=[
            pl.BlockSpec((TB, H), lambda i: (i, 0)),
            pl.BlockSpec((2, H), lambda i: (0, 0)),
        ],
        out_shape=[
            jax.ShapeDtypeStruct((N, H), jnp.float32),
            jax.ShapeDtypeStruct((2, H), jnp.float32),
        ],
    )(acc2, hp, WlT, bl2, WrT)


def _bn_apply(pre, st, hp, g, be):
    mean = st[0] / N
    var = st[1] / N - mean * mean
    inv = lax.rsqrt(var + 1e-5)
    outv = (pre - mean) * (inv * g) + be
    return jnp.maximum(outv, 0.0) + hp[:, :H]


def _passB_body(pre_r, st_r, hp_r, g_r, be_r, hp_o):
    hp_o[...] = _pack(_bn_apply(pre_r[...], st_r[...], hp_r[...],
                                g_r[...], be_r[...]))


def _passB(pre, st, hp, g2, be2):
    return pl.pallas_call(
        _passB_body,
        grid=(GRID,),
        in_specs=[
            pl.BlockSpec((TB, H), lambda i: (i, 0)),
            pl.BlockSpec((2, H), lambda i: (0, 0)),
            pl.BlockSpec((TB, PW), lambda i: (i, 0)),
            pl.BlockSpec((1, H), lambda i: (0, 0)),
            pl.BlockSpec((1, H), lambda i: (0, 0)),
        ],
        out_specs=pl.BlockSpec((TB, PW), lambda i: (i, 0)),
        out_shape=jax.ShapeDtypeStruct((N, PW), jnp.float32),
    )(pre, st, hp, g2, be2)


def _passB3_body(pre_r, st_r, hp_r, g_r, be_r, wc_r, bc_r, out_r):
    hnew = _bn_apply(pre_r[...], st_r[...], hp_r[...], g_r[...], be_r[...])
    out_r[...] = (jnp.dot(hnew, wc_r[...], preferred_element_type=jnp.float32)
                  + bc_r[...])


def _passB3(pre, st, hp, g2, be2, WcT, bc2):
    return pl.pallas_call(
        _passB3_body,
        grid=(GRID,),
        in_specs=[
            pl.BlockSpec((TB, H), lambda i: (i, 0)),
            pl.BlockSpec((2, H), lambda i: (0, 0)),
            pl.BlockSpec((TB, PW), lambda i: (i, 0)),
            pl.BlockSpec((1, H), lambda i: (0, 0)),
            pl.BlockSpec((1, H), lambda i: (0, 0)),
            pl.BlockSpec((H, C), lambda i: (0, 0)),
            pl.BlockSpec((1, C), lambda i: (0, 0)),
        ],
        out_specs=pl.BlockSpec((TB, C), lambda i: (i, 0)),
        out_shape=jax.ShapeDtypeStruct((N, C), jnp.float32),
    )(pre, st, hp, g2, be2, WcT, bc2)


def kernel(x, edge_index, W_in, b_in, Wl1, bl1, Wr1, g1, be1,
           Wl2, bl2, Wr2, g2, be2, Wl3, bl3, Wr3, g3, be3,
           W_cls, b_cls):
    src = edge_index[0]
    dst = edge_index[1]
    srcp = jnp.concatenate([src, jnp.zeros((E_PAD - E,), jnp.int32)])
    dstp = jnp.concatenate([dst, jnp.full((E_PAD - E,), NROWS - 1, jnp.int32)])
    shard_id = dstp // SHARD
    local = dstp - shard_id * SHARD
    dst4 = jnp.stack([jnp.where(shard_id == p, local, LTRASH)
                      for p in range(SHARDS)])

    hp = _k0(x, W_in.T, b_in.reshape(1, H))

    blocks = [(Wl1, bl1, Wr1, g1, be1), (Wl2, bl2, Wr2, g2, be2),
              (Wl3, bl3, Wr3, g3, be3)]
    out = None
    for k, (Wl, bl, Wr, g, be) in enumerate(blocks):
        acc2 = _segsum(hp, srcp, dst4)
        pre, st = _passA(acc2, hp, Wl.T, bl.reshape(1, H), Wr.T)
        if k < 2:
            hp = _passB(pre, st, hp, g.reshape(1, H), be.reshape(1, H))
        else:
            out = _passB3(pre, st, hp, g.reshape(1, H), be.reshape(1, H),
                          W_cls.T, b_cls.reshape(1, C))
    return out


# speedup vs baseline: 2.5990x; 2.5990x over previous
import jax
import jax.numpy as jnp
from jax import lax
from jax.experimental import pallas as pl
from jax.experimental.pallas import tpu as pltpu
from jax.experimental.pallas import tpu_sc as plsc

N = 50000
E = 800000
D = 128
H = 64
C = 2
PW = 128

NC = 2
NS = 16
NW = NC * NS
CB = 64
CH = 395
E_PAD = NC * NS * CH * CB
SHARDS = 4
SHARD = 13824
NROWS = SHARDS * SHARD
ACC_ROWS = SHARD + 128
LTRASH = SHARD
ZPT = ACC_ROWS // NS
ZR = 8
WPT = SHARD // NS
TB = 1000
GRID = N // TB

_SC_MESH = plsc.VectorSubcoreMesh(core_axis_name="c", subcore_axis_name="s")


def _segsum_body(hp, srcp, dst4, bnds, out, acc, zbuf, bnd_v, src_v, dst_v,
                 rows, sg0, sg1):
    c = lax.axis_index("c")
    s = lax.axis_index("s")
    w = s * NC + c
    zv = jnp.zeros((16,), jnp.float32)

    pltpu.sync_copy(bnds, bnd_v)

    def zrow(i, _):
        for k in range(0, PW, 16):
            zbuf[i, pl.ds(k, 16)] = zv
        return 0

    lax.fori_loop(0, ZR, zrow, 0)

    def shard_pass(p):
        for j in range(ZPT // ZR):
            pltpu.sync_copy(zbuf, acc.at[pl.ds(s * ZPT + j * ZR, ZR)])
        plsc.subcore_barrier()

        b16 = bnd_v[...]
        glo = b16[2 * p]
        ghi = b16[2 * p + 1]

        def pred(chunk):
            g = chunk * NW + w
            return jnp.logical_and(g >= glo, g < ghi)

        def fire(chunk, b):
            base = (chunk * NW + w) * CB
            sem = sg0 if b == 0 else sg1

            @pl.when(pred(chunk))
            def _():
                pltpu.sync_copy(srcp.at[pl.ds(base, CB)], src_v.at[b])
                pltpu.sync_copy(dst4.at[p, pl.ds(base, CB)], dst_v.at[b])
                pltpu.async_copy(hp.at[src_v.at[b]], rows.at[b], sem)

        def drain(chunk, b):
            sem = sg0 if b == 0 else sg1

            @pl.when(pred(chunk))
            def _():
                pltpu.make_async_copy(hp.at[src_v.at[b]], rows.at[b],
                                      sem).wait()
                pltpu.sync_copy(rows.at[b], acc.at[dst_v.at[b]], add=True)

        fire(0, 0)

        def body(i, _):
            e = 2 * i
            fire(e + 1, 1)
            drain(e, 0)
            fire(e + 2, 0)
            drain(e + 1, 1)
            return 0

        lax.fori_loop(0, (CH - 1) // 2, body, 0)
        drain(CH - 1, 0)

        plsc.subcore_barrier()
        pltpu.sync_copy(acc.at[pl.ds(s * WPT, WPT)],
                        out.at[c, pl.ds(p * SHARD + s * WPT, WPT)])
        plsc.subcore_barrier()

    for p in range(SHARDS):
        shard_pass(p)


def _segsum(hp, srcp, dst4, bnds):
    return pl.kernel(
        _segsum_body,
        out_type=jax.ShapeDtypeStruct((NC, NROWS, PW), jnp.float32),
        mesh=_SC_MESH,
        scratch_types=[
            pltpu.VMEM_SHARED((ACC_ROWS, PW), jnp.float32),
            pltpu.VMEM((ZR, PW), jnp.float32),
            pltpu.VMEM((16,), jnp.int32),
            pltpu.VMEM((2, CB), jnp.int32),
            pltpu.VMEM((2, CB), jnp.int32),
            pltpu.VMEM((2, CB, PW), jnp.float32),
            pltpu.SemaphoreType.DMA,
            pltpu.SemaphoreType.DMA,
        ],
    )(hp, srcp, dst4, bnds)


def _pack(h):
    col = lax.broadcasted_iota(jnp.int32, (h.shape[0], PW), 1)
    z = jnp.zeros((h.shape[0], PW - H), jnp.float32)
    return jnp.concatenate([h, z], axis=1) + (col == H).astype(jnp.float32)


def _k0_body(x_r, w_r, b_r, hp_r):
    h = jnp.dot(x_r[...], w_r[...], preferred_element_type=jnp.float32)
    hp_r[...] = _pack(jnp.maximum(h + b_r[...], 0.0))


def _k0(x, WinT, b_in2):
    return pl.pallas_call(
        _k0_body,
        grid=(GRID,),
        in_specs=[
            pl.BlockSpec((TB, D), lambda i: (i, 0)),
            pl.BlockSpec((D, H), lambda i: (0, 0)),
            pl.BlockSpec((1, H), lambda i: (0, 0)),
        ],
        out_specs=pl.BlockSpec((TB, PW), lambda i: (i, 0)),
        out_shape=jax.ShapeDtypeStruct((N, PW), jnp.float32),
    )(x, WinT, b_in2)


def _passA_body(acc_r, hp_r, wl_r, bl_r, wr_r, pre_r, st_r):
    i = pl.program_id(0)
    a = acc_r[0] + acc_r[1]
    deg = a[:, H:H + 1]
    agg = a[:, :H] / jnp.maximum(deg, 1.0)
    h = hp_r[:, :H]
    pre = (jnp.dot(agg, wl_r[...], preferred_element_type=jnp.float32)
           + bl_r[...]
           + jnp.dot(h, wr_r[...], preferred_element_type=jnp.float32))
    pre_r[...] = pre

    @pl.when(i == 0)
    def _():
        st_r[...] = jnp.zeros_like(st_r)

    st_r[...] += jnp.stack([jnp.sum(pre, axis=0), jnp.sum(pre * pre, axis=0)])


def _passA(acc2, hp, WlT, bl2, WrT):
    return pl.pallas_call(
        _passA_body,
        grid=(GRID,),
        in_specs=[
            pl.BlockSpec((NC, TB, PW), lambda i: (0, i, 0)),
            pl.BlockSpec((TB, PW), lambda i: (i, 0)),
            pl.BlockSpec((H, H), lambda i: (0, 0)),
            pl.BlockSpec((1, H), lambda i: (0, 0)),
            pl.BlockSpec((H, H), lambda i: (0, 0)),
        ],
        out_specs=[
            pl.BlockSpec((TB, H), lambda i: (i, 0)),
            pl.BlockSpec((2, H), lambda i: (0, 0)),
        ],
        out_shape=[
            jax.ShapeDtypeStruct((N, H), jnp.float32),
            jax.ShapeDtypeStruct((2, H), jnp.float32),
        ],
    )(acc2, hp, WlT, bl2, WrT)


def _bn_apply(pre, st, hp, g, be):
    mean = st[0] / N
    var = st[1] / N - mean * mean
    inv = lax.rsqrt(var + 1e-5)
    outv = (pre - mean) * (inv * g) + be
    return jnp.maximum(outv, 0.0) + hp[:, :H]


def _passB_body(pre_r, st_r, hp_r, g_r, be_r, hp_o):
    hp_o[...] = _pack(_bn_apply(pre_r[...], st_r[...], hp_r[...],
                                g_r[...], be_r[...]))


def _passB(pre, st, hp, g2, be2):
    return pl.pallas_call(
        _passB_body,
        grid=(GRID,),
        in_specs=[
            pl.BlockSpec((TB, H), lambda i: (i, 0)),
            pl.BlockSpec((2, H), lambda i: (0, 0)),
            pl.BlockSpec((TB, PW), lambda i: (i, 0)),
            pl.BlockSpec((1, H), lambda i: (0, 0)),
            pl.BlockSpec((1, H), lambda i: (0, 0)),
        ],
        out_specs=pl.BlockSpec((TB, PW), lambda i: (i, 0)),
        out_shape=jax.ShapeDtypeStruct((N, PW), jnp.float32),
    )(pre, st, hp, g2, be2)


def _passB3_body(pre_r, st_r, hp_r, g_r, be_r, wc_r, bc_r, out_r):
    hnew = _bn_apply(pre_r[...], st_r[...], hp_r[...], g_r[...], be_r[...])
    out_r[...] = (jnp.dot(hnew, wc_r[...], preferred_element_type=jnp.float32)
                  + bc_r[...])


def _passB3(pre, st, hp, g2, be2, WcT, bc2):
    return pl.pallas_call(
        _passB3_body,
        grid=(GRID,),
        in_specs=[
            pl.BlockSpec((TB, H), lambda i: (i, 0)),
            pl.BlockSpec((2, H), lambda i: (0, 0)),
            pl.BlockSpec((TB, PW), lambda i: (i, 0)),
            pl.BlockSpec((1, H), lambda i: (0, 0)),
            pl.BlockSpec((1, H), lambda i: (0, 0)),
            pl.BlockSpec((H, C), lambda i: (0, 0)),
            pl.BlockSpec((1, C), lambda i: (0, 0)),
        ],
        out_specs=pl.BlockSpec((TB, C), lambda i: (i, 0)),
        out_shape=jax.ShapeDtypeStruct((N, C), jnp.float32),
    )(pre, st, hp, g2, be2, WcT, bc2)


def kernel(x, edge_index, W_in, b_in, Wl1, bl1, Wr1, g1, be1,
           Wl2, bl2, Wr2, g2, be2, Wl3, bl3, Wr3, g3, be3,
           W_cls, b_cls):
    src = edge_index[0]
    dst = edge_index[1]
    sid = dst // SHARD
    order = jnp.argsort(sid)
    src_s = src[order]
    dst_s = dst[order]
    srcp = jnp.concatenate([src_s, jnp.zeros((E_PAD - E,), jnp.int32)])
    dstp = jnp.concatenate([dst_s,
                            jnp.full((E_PAD - E,), NROWS - 1, jnp.int32)])
    shard_id = dstp // SHARD
    local = dstp - shard_id * SHARD
    dst4 = jnp.stack([jnp.where(shard_id == p, local, LTRASH)
                      for p in range(SHARDS)])
    off = jnp.concatenate([
        jnp.zeros((1,), jnp.int32),
        jnp.searchsorted(jnp.sort(sid), jnp.arange(1, SHARDS,
                                                   dtype=jnp.int32)
                         ).astype(jnp.int32),
        jnp.full((1,), E_PAD, jnp.int32),
    ])
    glo = off[:SHARDS] // CB
    ghi = (off[1:] + CB - 1) // CB
    bnds = jnp.zeros((16,), jnp.int32)
    bnds = bnds.at[0:2 * SHARDS:2].set(glo).at[1:2 * SHARDS:2].set(ghi)

    hp = _k0(x, W_in.T, b_in.reshape(1, H))

    blocks = [(Wl1, bl1, Wr1, g1, be1), (Wl2, bl2, Wr2, g2, be2),
              (Wl3, bl3, Wr3, g3, be3)]
    out = None
    for k, (Wl, bl, Wr, g, be) in enumerate(blocks):
        acc2 = _segsum(hp, srcp, dst4, bnds)
        pre, st = _passA(acc2, hp, Wl.T, bl.reshape(1, H), Wr.T)
        if k < 2:
            hp = _passB(pre, st, hp, g.reshape(1, H), be.reshape(1, H))
        else:
            out = _passB3(pre, st, hp, g.reshape(1, H), be.reshape(1, H),
                          W_cls.T, b_cls.reshape(1, C))
    return out
